# Initial kernel scaffold; baseline (speedup 1.0000x reference)
import jax
import jax.numpy as jnp
from jax import lax
from jax.experimental import pallas as pl
from jax.experimental.pallas import tpu as pltpu
from jax.experimental.pallas import tpu_sc as plsc

N = 10000
E = 320000
B = 64
D = 128
NC = 2
NS = 16
NW = NC * NS
FPW = D // NW
EC = 4000
NEC = E // EC
WC = 1000
NWC = N // WC
SLAB = N * FPW

_MESH = dict(core_axis_name="c", subcore_axis_name="s",
             num_cores=NC, num_subcores=NS)


def _wid():
    return lax.axis_index("s") * NC + lax.axis_index("c")


def _zero_slab(ref, nwords):
    z = jnp.zeros((16,), jnp.float32)

    def body(i, carry):
        ref[pl.ds(i * 16, 16)] = z
        return carry

    lax.fori_loop(0, nwords // 16, body, 0)


def _unpack(pw):
    src = lax.shift_right_logical(pw, 14)
    dst = jnp.bitwise_and(pw, 16383)
    return src, dst


def _edge_chunk_loop(packed_hbm, ebuf0, ebuf1, esem0, esem1, per_vec,
                     extra_start=None, extra_wait=None):
    def start(ci, which):
        buf, sem = (ebuf0, esem0) if which == 0 else (ebuf1, esem1)
        pltpu.async_copy(packed_hbm.at[pl.ds(ci * EC, EC)], buf, sem)
        if extra_start is not None:
            extra_start(ci, which)

    def wait(which):
        buf, sem = (ebuf0, esem0) if which == 0 else (ebuf1, esem1)
        pltpu.make_async_copy(packed_hbm.at[pl.ds(0, EC)], buf, sem).wait()
        if extra_wait is not None:
            extra_wait(which)

    start(0, 0)

    def chunk_body(p, carry):
        wait(0)
        start(2 * p + 1, 1)

        def vec0(v, c):
            per_vec(v, 0)
            return c

        lax.fori_loop(0, EC // 16, vec0, 0)
        wait(1)

        @pl.when(p < NEC // 2 - 1)
        def _():
            start(2 * p + 2, 0)

        def vec1(v, c):
            per_vec(v, 1)
            return c

        lax.fori_loop(0, EC // 16, vec1, 0)
        return carry

    lax.fori_loop(0, NEC // 2, chunk_body, 0)


def _gine_body(with_deg, h_hbm, packed_hbm, eaT_hbm, ewb_hbm, ebb_hbm,
               agg_hbm, deg_hbm, h_v, acc_v, deg_v, ebuf0, ebuf1,
               eabuf0, eabuf1, ew_s, eb_s, esem0, esem1, easem0, easem1):
    w = _wid()
    pltpu.sync_copy(h_hbm.at[w], h_v)
    pltpu.sync_copy(ewb_hbm.at[w], ew_s)
    pltpu.sync_copy(ebb_hbm.at[w], eb_s)
    _zero_slab(acc_v, SLAB)
    ew = [[ew_s[k, f] for f in range(FPW)] for k in range(4)]
    eb = [eb_s[f] for f in range(FPW)]

    def extra_start(ci, which):
        buf, sem = (eabuf0, easem0) if which == 0 else (eabuf1, easem1)
        pltpu.async_copy(eaT_hbm.at[:, pl.ds(ci * EC, EC)], buf, sem)

    def extra_wait(which):
        buf, sem = (eabuf0, easem0) if which == 0 else (eabuf1, easem1)
        pltpu.make_async_copy(eaT_hbm.at[:, pl.ds(0, EC)], buf, sem).wait()

    def per_vec(v, which):
        buf = ebuf0 if which == 0 else ebuf1
        eab = eabuf0 if which == 0 else eabuf1
        pw = buf[pl.ds(v * 16, 16)]
        src, dst = _unpack(pw)
        fsrc = lax.shift_left(src, 2)
        fdst = lax.shift_left(dst, 2)
        a = [eab[k, pl.ds(v * 16, 16)] for k in range(4)]
        for f in range(FPW):
            ea = a[0] * ew[0][f] + a[1] * ew[1][f] + \
                a[2] * ew[2][f] + a[3] * ew[3][f] + eb[f]
            hv = plsc.load_gather(h_v, [fsrc + f])
            m = jnp.maximum(hv + ea, 0.0)
            plsc.addupdate_scatter(acc_v, [fdst + f], m)

    _edge_chunk_loop(packed_hbm, ebuf0, ebuf1, esem0, esem1, per_vec,
                     extra_start, extra_wait)
    pltpu.sync_copy(acc_v, agg_hbm.at[w])

    if with_deg:
        @pl.when(w == 0)
        def _():
            _zero_slab(deg_v, N)
            ones = jnp.ones((16,), jnp.float32)

            def per_vec_deg(v, which):
                buf = ebuf0 if which == 0 else ebuf1
                _, dst = _unpack(buf[pl.ds(v * 16, 16)])
                plsc.addupdate_scatter(deg_v, [dst], ones)

            _edge_chunk_loop(packed_hbm, ebuf0, ebuf1, esem0, esem1,
                             per_vec_deg)
            pltpu.sync_copy(deg_v, deg_hbm)


def _make_gine(with_deg):
    out_type = [jax.ShapeDtypeStruct((NW, SLAB), jnp.float32)]
    if with_deg:
        out_type.append(jax.ShapeDtypeStruct((N,), jnp.float32))

    def body(*refs):
        if with_deg:
            (h_hbm, packed_hbm, eaT_hbm, ewb_hbm, ebb_hbm, agg_hbm, deg_hbm,
             *rest) = refs
        else:
            (h_hbm, packed_hbm, eaT_hbm, ewb_hbm, ebb_hbm, agg_hbm,
             *rest) = refs
            deg_hbm = None
        _gine_body(with_deg, h_hbm, packed_hbm, eaT_hbm, ewb_hbm, ebb_hbm,
                   agg_hbm, deg_hbm, *rest)

    return pl.kernel(
        body,
        out_type=tuple(out_type),
        mesh=plsc.VectorSubcoreMesh(**_MESH),
        scratch_types=[
            pltpu.VMEM((SLAB,), jnp.float32),
            pltpu.VMEM((SLAB,), jnp.float32),
            pltpu.VMEM((N,), jnp.float32),
            pltpu.VMEM((EC,), jnp.int32),
            pltpu.VMEM((EC,), jnp.int32),
            pltpu.VMEM((4, EC), jnp.float32),
            pltpu.VMEM((4, EC), jnp.float32),
            pltpu.SMEM((4, FPW), jnp.float32),
            pltpu.SMEM((FPW,), jnp.float32),
            pltpu.SemaphoreType.DMA,
            pltpu.SemaphoreType.DMA,
            pltpu.SemaphoreType.DMA,
            pltpu.SemaphoreType.DMA,
        ],
        name="gine_edge_pass" + ("_deg" if with_deg else ""),
    )


def _appnp_body(u0_hbm, packed_hbm, w_hbm, uout_hbm,
                u_v, acc_v, w_v, ebuf0, ebuf1, nbuf0, nbuf1,
                esem0, esem1, nsem0, nsem1):
    w = _wid()
    pltpu.sync_copy(u0_hbm.at[w], u_v)
    pltpu.sync_copy(w_hbm, w_v)
    _zero_slab(acc_v, SLAB)
    qvec = lax.shift_right_logical(lax.iota(jnp.int32, 16), 2)

    def per_vec(v, which):
        buf = ebuf0 if which == 0 else ebuf1
        pw = buf[pl.ds(v * 16, 16)]
        src, dst = _unpack(pw)
        fsrc = lax.shift_left(src, 2)
        fdst = lax.shift_left(dst, 2)
        for f in range(FPW):
            vals = plsc.load_gather(u_v, [fsrc + f])
            plsc.addupdate_scatter(acc_v, [fdst + f], vals)

    zero16 = jnp.zeros((16,), jnp.float32)

    def nstart(ci, which):
        buf, sem = (nbuf0, nsem0) if which == 0 else (nbuf1, nsem1)
        pltpu.async_copy(u0_hbm.at[w, pl.ds(ci * WC * FPW, WC * FPW)],
                         buf, sem)

    def nwait(which):
        buf, sem = (nbuf0, nsem0) if which == 0 else (nbuf1, nsem1)
        pltpu.make_async_copy(u0_hbm.at[w, pl.ds(0, WC * FPW)], buf,
                              sem).wait()

    def node_vec(ni, v, buf):
        base = ni * (WC * FPW) + v * 16
        uold = u_v[pl.ds(base, 16)]
        a = acc_v[pl.ds(base, 16)]
        u0v = buf[pl.ds(v * 16, 16)]
        widx = qvec + (base >> 2)
        wv = plsc.load_gather(w_v, [widx])
        u_v[pl.ds(base, 16)] = wv * (a + uold) + 0.1 * u0v
        acc_v[pl.ds(base, 16)] = zero16

    def iter_body(t, carry):
        _edge_chunk_loop(packed_hbm, ebuf0, ebuf1, esem0, esem1, per_vec)
        nstart(0, 0)

        def nchunk(p, c):
            nwait(0)
            nstart(2 * p + 1, 1)

            def nv0(v, d):
                node_vec(2 * p, v, nbuf0)
                return d

            lax.fori_loop(0, WC * FPW // 16, nv0, 0)
            nwait(1)

            @pl.when(p < NWC // 2 - 1)
            def _():
                nstart(2 * p + 2, 0)

            def nv1(v, d):
                node_vec(2 * p + 1, v, nbuf1)
                return d

            lax.fori_loop(0, WC * FPW // 16, nv1, 0)
            return c

        lax.fori_loop(0, NWC // 2, nchunk, 0)
        return carry

    lax.fori_loop(0, 16, iter_body, 0)
    pltpu.sync_copy(u_v, uout_hbm.at[w])


_appnp = pl.kernel(
    _appnp_body,
    out_type=jax.ShapeDtypeStruct((NW, SLAB), jnp.float32),
    mesh=plsc.VectorSubcoreMesh(**_MESH),
    scratch_types=[
        pltpu.VMEM((SLAB,), jnp.float32),
        pltpu.VMEM((SLAB,), jnp.float32),
        pltpu.VMEM((N,), jnp.float32),
        pltpu.VMEM((EC,), jnp.int32),
        pltpu.VMEM((EC,), jnp.int32),
        pltpu.VMEM((WC * FPW,), jnp.float32),
        pltpu.VMEM((WC * FPW,), jnp.float32),
        pltpu.SemaphoreType.DMA,
        pltpu.SemaphoreType.DMA,
        pltpu.SemaphoreType.DMA,
        pltpu.SemaphoreType.DMA,
    ],
    name="appnp16",
)


def _gelu(x):
    return jax.nn.gelu(x, approximate=False)


def _ln(x, g, b):
    mu = jnp.mean(x, axis=-1, keepdims=True)
    var = jnp.mean((x - mu) ** 2, axis=-1, keepdims=True)
    return (x - mu) / jnp.sqrt(var + 1e-5) * g + b


RB = 1000


def _mlp1_kernel(h_ref, a_ref, w1_ref, b1_ref, w2_ref, b2_ref, g_ref,
                 bb_ref, out_ref):
    z = h_ref[...] + a_ref[...]
    t = _gelu(jnp.dot(z, w1_ref[...], preferred_element_type=jnp.float32)
              + b1_ref[...])
    y = jnp.dot(t, w2_ref[...], preferred_element_type=jnp.float32) \
        + b2_ref[...]
    out_ref[...] = _ln(_gelu(y), g_ref[...], bb_ref[...])


def _mlp2_kernel(h_ref, a_ref, deg_ref, w1_ref, b1_ref, w2_ref, b2_ref,
                 g_ref, bb_ref, u0_ref, w_ref, sdeg_ref):
    z = h_ref[...] + a_ref[...]
    t = _gelu(jnp.dot(z, w1_ref[...], preferred_element_type=jnp.float32)
              + b1_ref[...])
    y = jnp.dot(t, w2_ref[...], preferred_element_type=jnp.float32) \
        + b2_ref[...]
    h2 = _ln(y, g_ref[...], bb_ref[...])
    degt = deg_ref[...] + 1.0
    dis = lax.rsqrt(degt)
    u0_ref[...] = h2 * dis
    w_ref[...] = 0.9 / degt
    sdeg_ref[...] = jnp.sqrt(degt)


def _pool_kernel(u16_ref, sdeg_ref, batch_ref, w1_ref, b1_ref, w2_ref,
                 b2_ref, h_ref, g_ref):
    h = u16_ref[...] * sdeg_ref[...]
    t = _gelu(jnp.dot(h, w1_ref[...], preferred_element_type=jnp.float32)
              + b1_ref[...])
    gate = jnp.dot(t, w2_ref[...], preferred_element_type=jnp.float32) \
        + b2_ref[...]
    iot = lax.broadcasted_iota(jnp.int32, (1, B), 1)
    mask = batch_ref[...] == iot
    oh = mask.astype(jnp.float32)
    gmax = jnp.max(jnp.where(mask, gate, -3.0e38), axis=0, keepdims=True)
    gm_n = jnp.sum(oh * gmax, axis=1, keepdims=True)
    ex = jnp.exp(gate - gm_n)
    den = lax.dot_general(oh, ex, (((0,), (0,)), ((), ())),
                          preferred_element_type=jnp.float32)
    den_n = jnp.sum(oh * jnp.reshape(den, (1, B)), axis=1, keepdims=True)
    att = ex / (den_n + 1e-16)
    h_ref[...] = h
    g_ref[...] = lax.dot_general(oh, att * h, (((0,), (0,)), ((), ())),
                                 preferred_element_type=jnp.float32)


def _row_block(i):
    return (i, 0)


def _full_block(i):
    return (0, 0)


def _mlp1_call(h, agg, w1, b1, w2, b2, g, bb):
    spec_r = pl.BlockSpec((RB, D), _row_block)
    spec_w = pl.BlockSpec((D, D), _full_block)
    spec_v = pl.BlockSpec((1, D), _full_block)
    return pl.pallas_call(
        _mlp1_kernel,
        grid=(N // RB,),
        in_specs=[spec_r, spec_r, spec_w, spec_v, spec_w, spec_v, spec_v,
                  spec_v],
        out_specs=spec_r,
        out_shape=jax.ShapeDtypeStruct((N, D), jnp.float32),
    )(h, agg, w1, b1.reshape(1, D), w2, b2.reshape(1, D),
      g.reshape(1, D), bb.reshape(1, D))


def _mlp2_call(h, agg, deg, w1, b1, w2, b2, g, bb):
    spec_r = pl.BlockSpec((RB, D), _row_block)
    spec_w = pl.BlockSpec((D, D), _full_block)
    spec_v = pl.BlockSpec((1, D), _full_block)
    spec_c = pl.BlockSpec((RB, 1), _row_block)
    return pl.pallas_call(
        _mlp2_kernel,
        grid=(N // RB,),
        in_specs=[spec_r, spec_r, spec_c, spec_w, spec_v, spec_w, spec_v,
                  spec_v, spec_v],
        out_specs=[spec_r, spec_c, spec_c],
        out_shape=[jax.ShapeDtypeStruct((N, D), jnp.float32),
                   jax.ShapeDtypeStruct((N, 1), jnp.float32),
                   jax.ShapeDtypeStruct((N, 1), jnp.float32)],
    )(h, agg, deg.reshape(N, 1), w1, b1.reshape(1, D), w2,
      b2.reshape(1, D), g.reshape(1, D), bb.reshape(1, D))


def _pool_call(u16, sdeg, batch, w1, b1, w2, b2):
    return pl.pallas_call(
        _pool_kernel,
        out_shape=[jax.ShapeDtypeStruct((N, D), jnp.float32),
                   jax.ShapeDtypeStruct((B, D), jnp.float32)],
    )(u16, sdeg, batch.reshape(N, 1), w1, b1.reshape(1, D // 2), w2,
      b2.reshape(1, 1))


def _to_blocked(x):
    return x.reshape(N, NW, FPW).transpose(1, 0, 2).reshape(NW, SLAB)


def _from_blocked(xb):
    return xb.reshape(NW, N, FPW).transpose(1, 0, 2).reshape(N, D)


def _block_ew(ew):
    return ew.reshape(4, NW, FPW).transpose(1, 0, 2)


def _block_eb(ebv):
    return ebv.reshape(NW, FPW)


def kernel(x, edge_index, batch, edge_attr, c1_w1, c1_b1, c1_w2, c1_b2,
           c1_ew, c1_eb, n1_g, n1_b, c2_w1, c2_b1, c2_w2, c2_b2, c2_ew,
           c2_eb, n2_g, n2_b, pg_w1, pg_b1, pg_w2, pg_b2):
    src = edge_index[0]
    dst = edge_index[1]
    packed = jnp.bitwise_or(lax.shift_left(src, 14), dst)
    eaT = edge_attr.T

    gine_deg = _make_gine(True)
    gine = _make_gine(False)

    xb = _to_blocked(x)
    agg1b, deg = gine_deg(xb, packed, eaT, _block_ew(c1_ew),
                          _block_eb(c1_eb))
    h1 = _mlp1_call(x, _from_blocked(agg1b), c1_w1, c1_b1, c1_w2, c1_b2,
                    n1_g, n1_b)

    agg2b, = gine(_to_blocked(h1), packed, eaT, _block_ew(c2_ew),
                  _block_eb(c2_eb))
    u0, wvec, sdeg = _mlp2_call(h1, _from_blocked(agg2b), deg, c2_w1,
                                c2_b1, c2_w2, c2_b2, n2_g, n2_b)

    u16b = _appnp(_to_blocked(u0), packed, wvec.reshape(N))
    h, g = _pool_call(_from_blocked(u16b), sdeg, batch, pg_w1, pg_b1,
                      pg_w2, pg_b2)
    return (h, g)


# speedup vs baseline: 3.1180x; 3.1180x over previous
import jax
import jax.numpy as jnp
from jax import lax
from jax.experimental import pallas as pl
from jax.experimental.pallas import tpu as pltpu
from jax.experimental.pallas import tpu_sc as plsc

N = 10000
E = 320000
B = 64
D = 128
NC = 2
NS = 16
NW = NC * NS
FPW = D // NW
EC = 4000
NEC = E // EC
WC = 1000
NWC = N // WC
SLAB = N * FPW

_MESH = dict(core_axis_name="c", subcore_axis_name="s",
             num_cores=NC, num_subcores=NS)


def _wid():
    return lax.axis_index("s") * NC + lax.axis_index("c")


def _zero_slab(ref, nwords):
    z = jnp.zeros((16,), jnp.float32)

    def body(i, carry):
        ref[pl.ds(i * 16, 16)] = z
        return carry

    lax.fori_loop(0, nwords // 16, body, 0)


def _unpack(pw):
    src = lax.shift_right_logical(pw, 14)
    dst = jnp.bitwise_and(pw, 16383)
    return src, dst


def _edge_chunk_loop(packed_hbm, ebuf0, ebuf1, esem0, esem1, per_vec,
                     extra_start=None, extra_wait=None):
    def start(ci, which):
        buf, sem = (ebuf0, esem0) if which == 0 else (ebuf1, esem1)
        pltpu.async_copy(packed_hbm.at[pl.ds(ci * EC, EC)], buf, sem)
        if extra_start is not None:
            extra_start(ci, which)

    def wait(which):
        buf, sem = (ebuf0, esem0) if which == 0 else (ebuf1, esem1)
        pltpu.make_async_copy(packed_hbm.at[pl.ds(0, EC)], buf, sem).wait()
        if extra_wait is not None:
            extra_wait(which)

    start(0, 0)

    def chunk_body(p, carry):
        wait(0)
        start(2 * p + 1, 1)

        def vec0(v, c):
            per_vec(v, 0)
            return c

        lax.fori_loop(0, EC // 16, vec0, 0)
        wait(1)

        @pl.when(p < NEC // 2 - 1)
        def _():
            start(2 * p + 2, 0)

        def vec1(v, c):
            per_vec(v, 1)
            return c

        lax.fori_loop(0, EC // 16, vec1, 0)
        return carry

    lax.fori_loop(0, NEC // 2, chunk_body, 0)


def _gine_body(with_deg, h_hbm, packed_hbm, eaT_hbm, wcoef_hbm,
               agg_hbm, deg_hbm, h_v, acc_v, deg_v, ebuf0, ebuf1,
               eabuf0, eabuf1, wcoef_v, esem0, esem1, easem0, easem1):
    w = _wid()
    pltpu.sync_copy(h_hbm.at[w], h_v)
    pltpu.sync_copy(wcoef_hbm.at[w], wcoef_v)
    _zero_slab(acc_v, SLAB)
    ewvec = wcoef_v[pl.ds(0, 16)]
    ebvec = wcoef_v[pl.ds(16, 16)]
    ew = [[ewvec[k * FPW + f] for f in range(FPW)] for k in range(4)]
    eb = [ebvec[f] for f in range(FPW)]

    def extra_start(ci, which):
        buf, sem = (eabuf0, easem0) if which == 0 else (eabuf1, easem1)
        pltpu.async_copy(eaT_hbm.at[:, pl.ds(ci * EC, EC)], buf, sem)

    def extra_wait(which):
        buf, sem = (eabuf0, easem0) if which == 0 else (eabuf1, easem1)
        pltpu.make_async_copy(eaT_hbm.at[:, pl.ds(0, EC)], buf, sem).wait()

    def per_vec(v, which):
        buf = ebuf0 if which == 0 else ebuf1
        eab = eabuf0 if which == 0 else eabuf1
        pw = buf[pl.ds(v * 16, 16)]
        src, dst = _unpack(pw)
        fsrc = lax.shift_left(src, 2)
        fdst = lax.shift_left(dst, 2)
        a = [eab[k, pl.ds(v * 16, 16)] for k in range(4)]
        for f in range(FPW):
            ea = a[0] * ew[0][f] + a[1] * ew[1][f] + \
                a[2] * ew[2][f] + a[3] * ew[3][f] + eb[f]
            hv = plsc.load_gather(h_v, [fsrc + f])
            m = jnp.maximum(hv + ea, 0.0)
            plsc.addupdate_scatter(acc_v, [fdst + f], m)

    _edge_chunk_loop(packed_hbm, ebuf0, ebuf1, esem0, esem1, per_vec,
                     extra_start, extra_wait)
    pltpu.sync_copy(acc_v, agg_hbm.at[w])

    if with_deg:
        @pl.when(w == 0)
        def _():
            _zero_slab(deg_v, N)
            ones = jnp.ones((16,), jnp.float32)

            def per_vec_deg(v, which):
                buf = ebuf0 if which == 0 else ebuf1
                _, dst = _unpack(buf[pl.ds(v * 16, 16)])
                plsc.addupdate_scatter(deg_v, [dst], ones)

            _edge_chunk_loop(packed_hbm, ebuf0, ebuf1, esem0, esem1,
                             per_vec_deg)
            pltpu.sync_copy(deg_v, deg_hbm)


def _make_gine(with_deg):
    out_type = [jax.ShapeDtypeStruct((NW, SLAB), jnp.float32)]
    if with_deg:
        out_type.append(jax.ShapeDtypeStruct((N,), jnp.float32))

    def body(*refs):
        if with_deg:
            (h_hbm, packed_hbm, eaT_hbm, wcoef_hbm, agg_hbm, deg_hbm,
             *rest) = refs
        else:
            (h_hbm, packed_hbm, eaT_hbm, wcoef_hbm, agg_hbm,
             *rest) = refs
            deg_hbm = None
        _gine_body(with_deg, h_hbm, packed_hbm, eaT_hbm, wcoef_hbm,
                   agg_hbm, deg_hbm, *rest)

    return pl.kernel(
        body,
        out_type=tuple(out_type),
        mesh=plsc.VectorSubcoreMesh(**_MESH),
        scratch_types=[
            pltpu.VMEM((SLAB,), jnp.float32),
            pltpu.VMEM((SLAB,), jnp.float32),
            pltpu.VMEM((N,), jnp.float32),
            pltpu.VMEM((EC,), jnp.int32),
            pltpu.VMEM((EC,), jnp.int32),
            pltpu.VMEM((4, EC), jnp.float32),
            pltpu.VMEM((4, EC), jnp.float32),
            pltpu.VMEM((32,), jnp.float32),
            pltpu.SemaphoreType.DMA,
            pltpu.SemaphoreType.DMA,
            pltpu.SemaphoreType.DMA,
            pltpu.SemaphoreType.DMA,
        ],
        compiler_params=pltpu.CompilerParams(use_tc_tiling_on_sc=False, needs_layout_passes=False),
        name="gine_edge_pass" + ("_deg" if with_deg else ""),
    )


def _appnp_body(u0_hbm, packed_hbm, w_hbm, uout_hbm,
                u_v, acc_v, w_v, ebuf0, ebuf1, nbuf0, nbuf1,
                esem0, esem1, nsem0, nsem1):
    w = _wid()
    pltpu.sync_copy(u0_hbm.at[w], u_v)
    pltpu.sync_copy(w_hbm, w_v)
    _zero_slab(acc_v, SLAB)
    qvec = lax.shift_right_logical(lax.iota(jnp.int32, 16), 2)

    def per_vec(v, which):
        buf = ebuf0 if which == 0 else ebuf1
        pw = buf[pl.ds(v * 16, 16)]
        src, dst = _unpack(pw)
        fsrc = lax.shift_left(src, 2)
        fdst = lax.shift_left(dst, 2)
        for f in range(FPW):
            vals = plsc.load_gather(u_v, [fsrc + f])
            plsc.addupdate_scatter(acc_v, [fdst + f], vals)

    zero16 = jnp.zeros((16,), jnp.float32)

    def nstart(ci, which):
        buf, sem = (nbuf0, nsem0) if which == 0 else (nbuf1, nsem1)
        pltpu.async_copy(u0_hbm.at[w, pl.ds(ci * WC * FPW, WC * FPW)],
                         buf, sem)

    def nwait(which):
        buf, sem = (nbuf0, nsem0) if which == 0 else (nbuf1, nsem1)
        pltpu.make_async_copy(u0_hbm.at[w, pl.ds(0, WC * FPW)], buf,
                              sem).wait()

    def node_vec(ni, v, buf):
        base = ni * (WC * FPW) + v * 16
        uold = u_v[pl.ds(base, 16)]
        a = acc_v[pl.ds(base, 16)]
        u0v = buf[pl.ds(v * 16, 16)]
        widx = qvec + (base >> 2)
        wv = plsc.load_gather(w_v, [widx])
        u_v[pl.ds(base, 16)] = wv * (a + uold) + 0.1 * u0v
        acc_v[pl.ds(base, 16)] = zero16

    def iter_body(t, carry):
        _edge_chunk_loop(packed_hbm, ebuf0, ebuf1, esem0, esem1, per_vec)
        nstart(0, 0)

        def nchunk(p, c):
            nwait(0)
            nstart(2 * p + 1, 1)

            def nv0(v, d):
                node_vec(2 * p, v, nbuf0)
                return d

            lax.fori_loop(0, WC * FPW // 16, nv0, 0)
            nwait(1)

            @pl.when(p < NWC // 2 - 1)
            def _():
                nstart(2 * p + 2, 0)

            def nv1(v, d):
                node_vec(2 * p + 1, v, nbuf1)
                return d

            lax.fori_loop(0, WC * FPW // 16, nv1, 0)
            return c

        lax.fori_loop(0, NWC // 2, nchunk, 0)
        return carry

    lax.fori_loop(0, 16, iter_body, 0)
    pltpu.sync_copy(u_v, uout_hbm.at[w])


_appnp = pl.kernel(
    _appnp_body,
    out_type=jax.ShapeDtypeStruct((NW, SLAB), jnp.float32),
    mesh=plsc.VectorSubcoreMesh(**_MESH),
    scratch_types=[
        pltpu.VMEM((SLAB,), jnp.float32),
        pltpu.VMEM((SLAB,), jnp.float32),
        pltpu.VMEM((N,), jnp.float32),
        pltpu.VMEM((EC,), jnp.int32),
        pltpu.VMEM((EC,), jnp.int32),
        pltpu.VMEM((WC * FPW,), jnp.float32),
        pltpu.VMEM((WC * FPW,), jnp.float32),
        pltpu.SemaphoreType.DMA,
        pltpu.SemaphoreType.DMA,
        pltpu.SemaphoreType.DMA,
        pltpu.SemaphoreType.DMA,
    ],
    compiler_params=pltpu.CompilerParams(use_tc_tiling_on_sc=False, needs_layout_passes=False),
    name="appnp16",
)


def _gelu(x):
    return 0.5 * x * (1.0 + lax.erf(x * 0.7071067811865476))


def _ln(x, g, b):
    mu = jnp.mean(x, axis=-1, keepdims=True)
    var = jnp.mean((x - mu) ** 2, axis=-1, keepdims=True)
    return (x - mu) / jnp.sqrt(var + 1e-5) * g + b


RB = 1000


def _mlp1_kernel(h_ref, a_ref, w1_ref, b1_ref, w2_ref, b2_ref, g_ref,
                 bb_ref, out_ref):
    z = h_ref[...] + a_ref[...]
    t = _gelu(jnp.dot(z, w1_ref[...], preferred_element_type=jnp.float32)
              + b1_ref[...])
    y = jnp.dot(t, w2_ref[...], preferred_element_type=jnp.float32) \
        + b2_ref[...]
    out_ref[...] = _ln(_gelu(y), g_ref[...], bb_ref[...])


def _mlp2_kernel(h_ref, a_ref, deg_ref, w1_ref, b1_ref, w2_ref, b2_ref,
                 g_ref, bb_ref, u0_ref, w_ref, sdeg_ref):
    z = h_ref[...] + a_ref[...]
    t = _gelu(jnp.dot(z, w1_ref[...], preferred_element_type=jnp.float32)
              + b1_ref[...])
    y = jnp.dot(t, w2_ref[...], preferred_element_type=jnp.float32) \
        + b2_ref[...]
    h2 = _ln(y, g_ref[...], bb_ref[...])
    degt = deg_ref[...] + 1.0
    dis = lax.rsqrt(degt)
    u0_ref[...] = h2 * dis
    w_ref[...] = 0.9 / degt
    sdeg_ref[...] = jnp.sqrt(degt)


def _pool_kernel(u16_ref, sdeg_ref, batch_ref, w1_ref, b1_ref, w2_ref,
                 b2_ref, h_ref, g_ref):
    h = u16_ref[...] * sdeg_ref[...]
    t = _gelu(jnp.dot(h, w1_ref[...], preferred_element_type=jnp.float32)
              + b1_ref[...])
    gate = jnp.dot(t, w2_ref[...], preferred_element_type=jnp.float32) \
        + b2_ref[...]
    iot = lax.broadcasted_iota(jnp.int32, (1, B), 1)
    mask = batch_ref[...] == iot
    oh = mask.astype(jnp.float32)
    gmax = jnp.max(jnp.where(mask, gate, -3.0e38), axis=0, keepdims=True)
    gm_n = jnp.sum(oh * gmax, axis=1, keepdims=True)
    ex = jnp.exp(gate - gm_n)
    den = lax.dot_general(oh, ex, (((0,), (0,)), ((), ())),
                          preferred_element_type=jnp.float32)
    den_n = jnp.sum(oh * jnp.reshape(den, (1, B)), axis=1, keepdims=True)
    att = ex / (den_n + 1e-16)
    h_ref[...] = h
    g_ref[...] = lax.dot_general(oh, att * h, (((0,), (0,)), ((), ())),
                                 preferred_element_type=jnp.float32)


def _row_block(i):
    return (i, 0)


def _full_block(i):
    return (0, 0)


def _mlp1_call(h, agg, w1, b1, w2, b2, g, bb):
    spec_r = pl.BlockSpec((RB, D), _row_block)
    spec_w = pl.BlockSpec((D, D), _full_block)
    spec_v = pl.BlockSpec((1, D), _full_block)
    return pl.pallas_call(
        _mlp1_kernel,
        grid=(N // RB,),
        in_specs=[spec_r, spec_r, spec_w, spec_v, spec_w, spec_v, spec_v,
                  spec_v],
        out_specs=spec_r,
        out_shape=jax.ShapeDtypeStruct((N, D), jnp.float32),
    )(h, agg, w1, b1.reshape(1, D), w2, b2.reshape(1, D),
      g.reshape(1, D), bb.reshape(1, D))


def _mlp2_call(h, agg, deg, w1, b1, w2, b2, g, bb):
    spec_r = pl.BlockSpec((RB, D), _row_block)
    spec_w = pl.BlockSpec((D, D), _full_block)
    spec_v = pl.BlockSpec((1, D), _full_block)
    spec_c = pl.BlockSpec((RB, 1), _row_block)
    return pl.pallas_call(
        _mlp2_kernel,
        grid=(N // RB,),
        in_specs=[spec_r, spec_r, spec_c, spec_w, spec_v, spec_w, spec_v,
                  spec_v, spec_v],
        out_specs=[spec_r, spec_c, spec_c],
        out_shape=[jax.ShapeDtypeStruct((N, D), jnp.float32),
                   jax.ShapeDtypeStruct((N, 1), jnp.float32),
                   jax.ShapeDtypeStruct((N, 1), jnp.float32)],
    )(h, agg, deg.reshape(N, 1), w1, b1.reshape(1, D), w2,
      b2.reshape(1, D), g.reshape(1, D), bb.reshape(1, D))


def _pool_call(u16, sdeg, batch, w1, b1, w2, b2):
    return pl.pallas_call(
        _pool_kernel,
        out_shape=[jax.ShapeDtypeStruct((N, D), jnp.float32),
                   jax.ShapeDtypeStruct((B, D), jnp.float32)],
    )(u16, sdeg, batch.reshape(N, 1), w1, b1.reshape(1, D // 2), w2,
      b2.reshape(1, 1))


def _to_blocked(x):
    return x.reshape(N, NW, FPW).transpose(1, 0, 2).reshape(NW, SLAB)


def _from_blocked(xb):
    return xb.reshape(NW, N, FPW).transpose(1, 0, 2).reshape(N, D)


def _wcoef(ew, ebv):
    ewb = ew.reshape(4, NW, FPW).transpose(1, 0, 2).reshape(NW, 16)
    ebb = ebv.reshape(NW, FPW)
    return jnp.concatenate(
        [ewb, ebb, jnp.zeros((NW, 12), jnp.float32)], axis=1)


def kernel(x, edge_index, batch, edge_attr, c1_w1, c1_b1, c1_w2, c1_b2,
           c1_ew, c1_eb, n1_g, n1_b, c2_w1, c2_b1, c2_w2, c2_b2, c2_ew,
           c2_eb, n2_g, n2_b, pg_w1, pg_b1, pg_w2, pg_b2):
    src = edge_index[0]
    dst = edge_index[1]
    packed = jnp.bitwise_or(lax.shift_left(src, 14), dst)
    eaT = edge_attr.T

    gine_deg = _make_gine(True)
    gine = _make_gine(False)

    xb = _to_blocked(x)
    agg1b, deg = gine_deg(xb, packed, eaT, _wcoef(c1_ew, c1_eb))
    h1 = _mlp1_call(x, _from_blocked(agg1b), c1_w1, c1_b1, c1_w2, c1_b2,
                    n1_g, n1_b)

    agg2b, = gine(_to_blocked(h1), packed, eaT, _wcoef(c2_ew, c2_eb))
    u0, wvec, sdeg = _mlp2_call(h1, _from_blocked(agg2b), deg, c2_w1,
                                c2_b1, c2_w2, c2_b2, n2_g, n2_b)

    u16b = _appnp(_to_blocked(u0), packed, wvec.reshape(N))
    h, g = _pool_call(_from_blocked(u16b), sdeg, batch, pg_w1, pg_b1,
                      pg_w2, pg_b2)
    return (h, g)


# speedup vs baseline: 11.6506x; 3.7365x over previous
import jax
import jax.numpy as jnp
from jax import lax
from jax.experimental import pallas as pl
from jax.experimental.pallas import tpu as pltpu
from jax.experimental.pallas import tpu_sc as plsc

N = 10000
E = 320000
B = 64
D = 128
NC = 2
NS = 16
NW = NC * NS
FPW = D // NW
EC = 4000
NEC = E // EC
EV = EC // 16
NV = N // 16

_MESH = dict(core_axis_name="c", subcore_axis_name="s",
             num_cores=NC, num_subcores=NS)

_Z16 = None


def _wid():
    return lax.axis_index("s") * NC + lax.axis_index("c")


def _zero_1d(ref, nwords, unroll=8):
    z = jnp.zeros((16,), jnp.float32)

    @plsc.parallel_loop(0, nwords // 16, unroll=unroll)
    def _(i):
        ref[pl.ds(i * 16, 16)] = z


def _zero_planes(ref):
    z = jnp.zeros((16,), jnp.float32)
    for f in range(FPW):
        @plsc.parallel_loop(0, NV, unroll=8)
        def _(i, _f=f):
            ref[_f, pl.ds(i * 16, 16)] = z


def _unpack(pw):
    src = lax.shift_right_logical(pw, 16)
    dst = jnp.bitwise_and(pw, 65535)
    return src, dst


def _edge_chunk_loop(packed_hbm, ebuf0, ebuf1, esem0, esem1, per_vec,
                     extra_start=None, extra_wait=None, unroll=5):
    def start(ci, which):
        buf, sem = (ebuf0, esem0) if which == 0 else (ebuf1, esem1)
        pltpu.async_copy(packed_hbm.at[pl.ds(ci * EC, EC)], buf, sem)
        if extra_start is not None:
            extra_start(ci, which)

    def wait(which):
        buf, sem = (ebuf0, esem0) if which == 0 else (ebuf1, esem1)
        pltpu.make_async_copy(packed_hbm.at[pl.ds(0, EC)], buf, sem).wait()
        if extra_wait is not None:
            extra_wait(which)

    start(0, 0)

    def chunk_body(p, carry):
        wait(0)
        start(2 * p + 1, 1)

        @plsc.parallel_loop(0, EV, unroll=unroll)
        def _(v):
            per_vec(v, 0)

        wait(1)

        @pl.when(p < NEC // 2 - 1)
        def _():
            start(2 * p + 2, 0)

        @plsc.parallel_loop(0, EV, unroll=unroll)
        def _(v):
            per_vec(v, 1)

        return carry

    lax.fori_loop(0, NEC // 2, chunk_body, 0)


def _gine_body(with_deg, h_hbm, packed_hbm, eaT_hbm, wcoef_hbm,
               agg_hbm, deg_hbm, h_v, acc_v, deg_v, ebuf0, ebuf1,
               eabuf0, eabuf1, wcoef_v, esem0, esem1, easem0, easem1):
    w = _wid()
    pltpu.sync_copy(h_hbm.at[w], h_v)
    pltpu.sync_copy(wcoef_hbm.at[w], wcoef_v)
    _zero_planes(acc_v)
    ewvec = wcoef_v[pl.ds(0, 16)]
    ebvec = wcoef_v[pl.ds(16, 16)]
    ew = [[ewvec[k * FPW + f] for f in range(FPW)] for k in range(4)]
    eb = [ebvec[f] for f in range(FPW)]

    def extra_start(ci, which):
        buf, sem = (eabuf0, easem0) if which == 0 else (eabuf1, easem1)
        pltpu.async_copy(eaT_hbm.at[:, pl.ds(ci * EC, EC)], buf, sem)

    def extra_wait(which):
        buf, sem = (eabuf0, easem0) if which == 0 else (eabuf1, easem1)
        pltpu.make_async_copy(eaT_hbm.at[:, pl.ds(0, EC)], buf, sem).wait()

    def per_vec(v, which):
        buf = ebuf0 if which == 0 else ebuf1
        eab = eabuf0 if which == 0 else eabuf1
        pw = buf[pl.ds(v * 16, 16)]
        src, dst = _unpack(pw)
        a = [eab[k, pl.ds(v * 16, 16)] for k in range(4)]
        for f in range(FPW):
            ea = a[0] * ew[0][f] + a[1] * ew[1][f] + \
                a[2] * ew[2][f] + a[3] * ew[3][f] + eb[f]
            hv = plsc.load_gather(h_v.at[f], [src])
            m = jnp.maximum(hv + ea, 0.0)
            plsc.addupdate_scatter(acc_v.at[f], [dst], m)

    _edge_chunk_loop(packed_hbm, ebuf0, ebuf1, esem0, esem1, per_vec,
                     extra_start, extra_wait, unroll=4)
    pltpu.sync_copy(acc_v, agg_hbm.at[w])

    if with_deg:
        @pl.when(w == 0)
        def _():
            _zero_1d(deg_v, N)
            ones = jnp.ones((16,), jnp.float32)

            def per_vec_deg(v, which):
                buf = ebuf0 if which == 0 else ebuf1
                _, dst = _unpack(buf[pl.ds(v * 16, 16)])
                plsc.addupdate_scatter(deg_v, [dst], ones)

            _edge_chunk_loop(packed_hbm, ebuf0, ebuf1, esem0, esem1,
                             per_vec_deg, unroll=8)
            pltpu.sync_copy(deg_v, deg_hbm)


def _make_gine(with_deg):
    out_type = [jax.ShapeDtypeStruct((NW, FPW, N), jnp.float32)]
    if with_deg:
        out_type.append(jax.ShapeDtypeStruct((N,), jnp.float32))

    def body(*refs):
        if with_deg:
            (h_hbm, packed_hbm, eaT_hbm, wcoef_hbm, agg_hbm, deg_hbm,
             *rest) = refs
        else:
            (h_hbm, packed_hbm, eaT_hbm, wcoef_hbm, agg_hbm,
             *rest) = refs
            deg_hbm = None
        _gine_body(with_deg, h_hbm, packed_hbm, eaT_hbm, wcoef_hbm,
                   agg_hbm, deg_hbm, *rest)

    return pl.kernel(
        body,
        out_type=tuple(out_type),
        mesh=plsc.VectorSubcoreMesh(**_MESH),
        scratch_types=[
            pltpu.VMEM((FPW, N), jnp.float32),
            pltpu.VMEM((FPW, N), jnp.float32),
            pltpu.VMEM((N,), jnp.float32),
            pltpu.VMEM((EC,), jnp.int32),
            pltpu.VMEM((EC,), jnp.int32),
            pltpu.VMEM((4, EC), jnp.float32),
            pltpu.VMEM((4, EC), jnp.float32),
            pltpu.VMEM((32,), jnp.float32),
            pltpu.SemaphoreType.DMA,
            pltpu.SemaphoreType.DMA,
            pltpu.SemaphoreType.DMA,
            pltpu.SemaphoreType.DMA,
        ],
        compiler_params=pltpu.CompilerParams(use_tc_tiling_on_sc=False, needs_layout_passes=False),
        name="gine_edge_pass" + ("_deg" if with_deg else ""),
    )


def _appnp_body(u0_hbm, packed_hbm, w_hbm, uout_hbm,
                u_v, acc_v, w_v, ebuf0, ebuf1, nbuf0, nbuf1,
                esem0, esem1, nsem0, nsem1):
    w = _wid()
    pltpu.sync_copy(u0_hbm.at[w], u_v)
    pltpu.sync_copy(w_hbm, w_v)
    _zero_planes(acc_v)

    def per_vec(v, which):
        buf = ebuf0 if which == 0 else ebuf1
        pw = buf[pl.ds(v * 16, 16)]
        src, dst = _unpack(pw)
        for f in range(FPW):
            vals = plsc.load_gather(u_v.at[f], [src])
            plsc.addupdate_scatter(acc_v.at[f], [dst], vals)

    def nstart(f, which):
        buf, sem = (nbuf0, nsem0) if which == 0 else (nbuf1, nsem1)
        pltpu.async_copy(u0_hbm.at[w, f], buf, sem)

    def nwait(which):
        buf, sem = (nbuf0, nsem0) if which == 0 else (nbuf1, nsem1)
        pltpu.make_async_copy(u0_hbm.at[w, 0], buf, sem).wait()

    z16 = jnp.zeros((16,), jnp.float32)

    def iter_body(t, carry):
        nstart(0, 0)
        nstart(1, 1)
        _edge_chunk_loop(packed_hbm, ebuf0, ebuf1, esem0, esem1, per_vec,
                         unroll=5)
        for f in range(FPW):
            nwait(f % 2)

            @plsc.parallel_loop(0, NV, unroll=5)
            def _(vv, _f=f):
                buf = nbuf0 if _f % 2 == 0 else nbuf1
                b = vv * 16
                uold = u_v[_f, pl.ds(b, 16)]
                a = acc_v[_f, pl.ds(b, 16)]
                u0v = buf[pl.ds(b, 16)]
                wv = w_v[pl.ds(b, 16)]
                u_v[_f, pl.ds(b, 16)] = wv * (a + uold) + 0.1 * u0v
                acc_v[_f, pl.ds(b, 16)] = z16

            if f + 2 < FPW:
                nstart(f + 2, f % 2)
        return carry

    lax.fori_loop(0, 16, iter_body, 0)
    pltpu.sync_copy(u_v, uout_hbm.at[w])


_appnp = pl.kernel(
    _appnp_body,
    out_type=jax.ShapeDtypeStruct((NW, FPW, N), jnp.float32),
    mesh=plsc.VectorSubcoreMesh(**_MESH),
    scratch_types=[
        pltpu.VMEM((FPW, N), jnp.float32),
        pltpu.VMEM((FPW, N), jnp.float32),
        pltpu.VMEM((N,), jnp.float32),
        pltpu.VMEM((EC,), jnp.int32),
        pltpu.VMEM((EC,), jnp.int32),
        pltpu.VMEM((N,), jnp.float32),
        pltpu.VMEM((N,), jnp.float32),
        pltpu.SemaphoreType.DMA,
        pltpu.SemaphoreType.DMA,
        pltpu.SemaphoreType.DMA,
        pltpu.SemaphoreType.DMA,
    ],
    compiler_params=pltpu.CompilerParams(use_tc_tiling_on_sc=False, needs_layout_passes=False),
    name="appnp16",
)


def _gelu(x):
    return 0.5 * x * (1.0 + lax.erf(x * 0.7071067811865476))


def _ln(x, g, b):
    mu = jnp.mean(x, axis=-1, keepdims=True)
    var = jnp.mean((x - mu) ** 2, axis=-1, keepdims=True)
    return (x - mu) / jnp.sqrt(var + 1e-5) * g + b


RB = 1000


def _mlp1_kernel(h_ref, a_ref, w1_ref, b1_ref, w2_ref, b2_ref, g_ref,
                 bb_ref, out_ref):
    z = h_ref[...] + a_ref[...]
    t = _gelu(jnp.dot(z, w1_ref[...], preferred_element_type=jnp.float32)
              + b1_ref[...])
    y = jnp.dot(t, w2_ref[...], preferred_element_type=jnp.float32) \
        + b2_ref[...]
    out_ref[...] = _ln(_gelu(y), g_ref[...], bb_ref[...])


def _mlp2_kernel(h_ref, a_ref, deg_ref, w1_ref, b1_ref, w2_ref, b2_ref,
                 g_ref, bb_ref, u0_ref, w_ref, sdeg_ref):
    z = h_ref[...] + a_ref[...]
    t = _gelu(jnp.dot(z, w1_ref[...], preferred_element_type=jnp.float32)
              + b1_ref[...])
    y = jnp.dot(t, w2_ref[...], preferred_element_type=jnp.float32) \
        + b2_ref[...]
    h2 = _ln(y, g_ref[...], bb_ref[...])
    degt = deg_ref[...] + 1.0
    dis = lax.rsqrt(degt)
    u0_ref[...] = h2 * dis
    w_ref[...] = 0.9 / degt
    sdeg_ref[...] = jnp.sqrt(degt)


def _pool_kernel(u16_ref, sdeg_ref, batch_ref, w1_ref, b1_ref, w2_ref,
                 b2_ref, h_ref, g_ref):
    h = u16_ref[...] * sdeg_ref[...]
    t = _gelu(jnp.dot(h, w1_ref[...], preferred_element_type=jnp.float32)
              + b1_ref[...])
    gate = jnp.dot(t, w2_ref[...], preferred_element_type=jnp.float32) \
        + b2_ref[...]
    iot = lax.broadcasted_iota(jnp.int32, (1, B), 1)
    mask = batch_ref[...] == iot
    oh = mask.astype(jnp.float32)
    gmax = jnp.max(jnp.where(mask, gate, -3.0e38), axis=0, keepdims=True)
    gm_n = jnp.sum(oh * gmax, axis=1, keepdims=True)
    ex = jnp.exp(gate - gm_n)
    den = lax.dot_general(oh, ex, (((0,), (0,)), ((), ())),
                          preferred_element_type=jnp.float32)
    den_n = jnp.sum(oh * jnp.reshape(den, (1, B)), axis=1, keepdims=True)
    att = ex / (den_n + 1e-16)
    h_ref[...] = h
    g_ref[...] = lax.dot_general(oh, att * h, (((0,), (0,)), ((), ())),
                                 preferred_element_type=jnp.float32)


def _row_block(i):
    return (i, 0)


def _full_block(i):
    return (0, 0)


def _mlp1_call(h, agg, w1, b1, w2, b2, g, bb):
    spec_r = pl.BlockSpec((RB, D), _row_block)
    spec_w = pl.BlockSpec((D, D), _full_block)
    spec_v = pl.BlockSpec((1, D), _full_block)
    return pl.pallas_call(
        _mlp1_kernel,
        grid=(N // RB,),
        in_specs=[spec_r, spec_r, spec_w, spec_v, spec_w, spec_v, spec_v,
                  spec_v],
        out_specs=spec_r,
        out_shape=jax.ShapeDtypeStruct((N, D), jnp.float32),
    )(h, agg, w1, b1.reshape(1, D), w2, b2.reshape(1, D),
      g.reshape(1, D), bb.reshape(1, D))


def _mlp2_call(h, agg, deg, w1, b1, w2, b2, g, bb):
    spec_r = pl.BlockSpec((RB, D), _row_block)
    spec_w = pl.BlockSpec((D, D), _full_block)
    spec_v = pl.BlockSpec((1, D), _full_block)
    spec_c = pl.BlockSpec((RB, 1), _row_block)
    return pl.pallas_call(
        _mlp2_kernel,
        grid=(N // RB,),
        in_specs=[spec_r, spec_r, spec_c, spec_w, spec_v, spec_w, spec_v,
                  spec_v, spec_v],
        out_specs=[spec_r, spec_c, spec_c],
        out_shape=[jax.ShapeDtypeStruct((N, D), jnp.float32),
                   jax.ShapeDtypeStruct((N, 1), jnp.float32),
                   jax.ShapeDtypeStruct((N, 1), jnp.float32)],
    )(h, agg, deg.reshape(N, 1), w1, b1.reshape(1, D), w2,
      b2.reshape(1, D), g.reshape(1, D), bb.reshape(1, D))


def _pool_call(u16, sdeg, batch, w1, b1, w2, b2):
    return pl.pallas_call(
        _pool_kernel,
        out_shape=[jax.ShapeDtypeStruct((N, D), jnp.float32),
                   jax.ShapeDtypeStruct((B, D), jnp.float32)],
    )(u16, sdeg, batch.reshape(N, 1), w1, b1.reshape(1, D // 2), w2,
      b2.reshape(1, 1))


def _to_blocked(x):
    return x.reshape(N, NW, FPW).transpose(1, 2, 0)


def _from_blocked(xb):
    return xb.transpose(2, 0, 1).reshape(N, D)


def _wcoef(ew, ebv):
    ewb = ew.reshape(4, NW, FPW).transpose(1, 0, 2).reshape(NW, 16)
    ebb = ebv.reshape(NW, FPW)
    return jnp.concatenate(
        [ewb, ebb, jnp.zeros((NW, 12), jnp.float32)], axis=1)


def kernel(x, edge_index, batch, edge_attr, c1_w1, c1_b1, c1_w2, c1_b2,
           c1_ew, c1_eb, n1_g, n1_b, c2_w1, c2_b1, c2_w2, c2_b2, c2_ew,
           c2_eb, n2_g, n2_b, pg_w1, pg_b1, pg_w2, pg_b2):
    src = edge_index[0]
    dst = edge_index[1]
    packed = jnp.bitwise_or(lax.shift_left(src, 16), dst)
    eaT = edge_attr.T

    gine_deg = _make_gine(True)
    gine = _make_gine(False)

    xb = _to_blocked(x)
    agg1b, deg = gine_deg(xb, packed, eaT, _wcoef(c1_ew, c1_eb))
    h1 = _mlp1_call(x, _from_blocked(agg1b), c1_w1, c1_b1, c1_w2, c1_b2,
                    n1_g, n1_b)

    agg2b, = gine(_to_blocked(h1), packed, eaT, _wcoef(c2_ew, c2_eb))
    u0, wvec, sdeg = _mlp2_call(h1, _from_blocked(agg2b), deg, c2_w1,
                                c2_b1, c2_w2, c2_b2, n2_g, n2_b)

    u16b = _appnp(_to_blocked(u0), packed, wvec.reshape(N))
    h, g = _pool_call(_from_blocked(u16b), sdeg, batch, pg_w1, pg_b1,
                      pg_w2, pg_b2)
    return (h, g)
